# baseline (speedup 1.0000x reference)
import functools

import jax
import jax.numpy as jnp
from jax import lax
from jax.experimental import pallas as pl
from jax.experimental.pallas import tpu as pltpu
from jax.experimental.pallas import tpu_sc as plsc

N = 10000
E = 320000
H = 128
H2 = 64
EPS = 1e-07


_BN = 2000
_BE = 4000


def _enc_node_body(x_ref, w_ref, b_ref, out_ref):
    h = jnp.dot(x_ref[...], w_ref[...], preferred_element_type=jnp.float32)
    h = h + b_ref[...]
    out_ref[0] = h[:, :H2]
    out_ref[1] = h[:, H2:]


def _enc_node(x, W, b):
    return pl.pallas_call(
        _enc_node_body,
        grid=(N // _BN,),
        in_specs=[
            pl.BlockSpec((_BN, 3), lambda i: (i, 0)),
            pl.BlockSpec((3, H), lambda i: (0, 0)),
            pl.BlockSpec((1, H), lambda i: (0, 0)),
        ],
        out_specs=pl.BlockSpec((2, _BN, H2), lambda i: (0, i, 0)),
        out_shape=jax.ShapeDtypeStruct((2, N, H2), jnp.float32),
    )(x, W, b)


def _enc_edge_body(a_ref, w_ref, b_ref, out_ref):
    h = jnp.dot(a_ref[...], w_ref[...], preferred_element_type=jnp.float32)
    h = h + b_ref[...]
    out_ref[0] = h[:, :H2]
    out_ref[1] = h[:, H2:]


def _enc_edge(attr, W, b):
    return pl.pallas_call(
        _enc_edge_body,
        grid=(E // _BE,),
        in_specs=[
            pl.BlockSpec((_BE, 7), lambda i: (i, 0)),
            pl.BlockSpec((7, H), lambda i: (0, 0)),
            pl.BlockSpec((1, H), lambda i: (0, 0)),
        ],
        out_specs=pl.BlockSpec((2, _BE, H2), lambda i: (0, i, 0)),
        out_shape=jax.ShapeDtypeStruct((2, E, H2), jnp.float32),
    )(attr, W, b)


def _mlp_body(relu_out, h_ref, a_ref, w_ref, b_ref, g_ref, be_ref, out_ref):
    a0 = a_ref[0, :, H2:] / (a_ref[0, :, :H2] + 1e-16)
    a1 = a_ref[1, :, H2:] / (a_ref[1, :, :H2] + 1e-16)
    hp = jnp.concatenate([h_ref[0] + a0, h_ref[1] + a1], axis=-1)
    t = jnp.dot(hp, w_ref[...], preferred_element_type=jnp.float32)
    t = t + b_ref[...]
    mu = jnp.mean(t, axis=-1, keepdims=True)
    var = jnp.mean((t - mu) * (t - mu), axis=-1, keepdims=True)
    y = (t - mu) / jnp.sqrt(var + 1e-5) * g_ref[...] + be_ref[...]
    if relu_out:
        y = jnp.maximum(y, 0.0)
        out_ref[0] = y[:, :H2]
        out_ref[1] = y[:, H2:]
    else:
        out_ref[...] = y


def _mlp(hs, aggr, Wc, bc, g, be, relu_out):
    if relu_out:
        out_spec = pl.BlockSpec((2, _BN, H2), lambda i: (0, i, 0))
        out_shape = jax.ShapeDtypeStruct((2, N, H2), jnp.float32)
    else:
        out_spec = pl.BlockSpec((_BN, H), lambda i: (i, 0))
        out_shape = jax.ShapeDtypeStruct((N, H), jnp.float32)
    return pl.pallas_call(
        functools.partial(_mlp_body, relu_out),
        grid=(N // _BN,),
        in_specs=[
            pl.BlockSpec((2, _BN, H2), lambda i: (0, i, 0)),
            pl.BlockSpec((2, _BN, H), lambda i: (0, i, 0)),
            pl.BlockSpec((H, H), lambda i: (0, 0)),
            pl.BlockSpec((1, H), lambda i: (0, 0)),
            pl.BlockSpec((1, H), lambda i: (0, 0)),
            pl.BlockSpec((1, H), lambda i: (0, 0)),
        ],
        out_specs=out_spec,
        out_shape=out_shape,
    )(hs, aggr, Wc, bc, g, be)


_NSUB = 16
_C = 80
_NCH = E // _C
_CPT = _NCH // _NSUB
_NPT = N // _NSUB
_FC = 25

_mesh = plsc.VectorSubcoreMesh(core_axis_name="c", subcore_axis_name="s")


def _conv_body(h_hbm, emb_hbm, idx_hbm, out_hbm,
               idxb0, idxb1, hrows0, hrows1, erows0, erows1, cbuf0, cbuf1,
               SW, sem_h0, sem_h1, sem_e0, sem_e1):
    cid = lax.axis_index("c")
    sid = lax.axis_index("s")
    cstart = sid * _CPT

    zero = jnp.zeros((16,), jnp.float32)

    def zbody(e, carry):
        for k in range(8):
            cbuf0[e, pl.ds(k * 16, 16)] = zero
        return carry

    lax.fori_loop(0, _C, zbody, 0, unroll=False)
    nz_full, nz_tail = divmod(_NPT, _C)
    for j in range(nz_full):
        nb = sid * _NPT + j * _C
        pltpu.sync_copy(cbuf0, SW.at[pl.ds(nb, _C)])
    if nz_tail:
        nb = sid * _NPT + nz_full * _C
        pltpu.sync_copy(cbuf0.at[pl.ds(0, nz_tail)], SW.at[pl.ds(nb, nz_tail)])
    plsc.subcore_barrier()

    def issue(ci, idxb, hrows, erows, sem_h, sem_e):
        cic = jnp.minimum(ci, _NCH - 1)
        pltpu.sync_copy(idx_hbm.at[cic], idxb)
        pltpu.async_copy(h_hbm.at[cid].at[idxb.at[0]], hrows, sem_h)
        pltpu.async_copy(emb_hbm.at[cid, pl.ds(cic * _C, _C)], erows, sem_e)

    def step(ci, idxb, hrows, erows, cbuf, sem_h, sem_e):
        pltpu.make_async_copy(h_hbm.at[cid].at[idxb.at[0]], hrows,
                              sem_h).wait()
        pltpu.make_async_copy(emb_hbm.at[cid, pl.ds(0, _C)], erows,
                              sem_e).wait()

        def comp(e, carry):
            for k in range(4):
                sl = pl.ds(k * 16, 16)
                msg = jnp.maximum(hrows[e, sl] + erows[e, sl], 0.0) + EPS
                ex = jnp.exp(msg)
                cbuf[e, sl] = ex
                cbuf[e, pl.ds(H2 + k * 16, 16)] = ex * msg
            return carry

        lax.fori_loop(0, _C, comp, 0, unroll=False)
        pltpu.sync_copy(cbuf, SW.at[idxb.at[1]], add=True)
        issue(ci + 2, idxb, hrows, erows, sem_h, sem_e)

    issue(cstart, idxb0, hrows0, erows0, sem_h0, sem_e0)
    issue(cstart + 1, idxb1, hrows1, erows1, sem_h1, sem_e1)

    def pair(i, carry):
        a = cstart + 2 * i
        step(a, idxb0, hrows0, erows0, cbuf0, sem_h0, sem_e0)
        step(a + 1, idxb1, hrows1, erows1, cbuf1, sem_h1, sem_e1)
        return carry

    lax.fori_loop(0, _CPT // 2, pair, 0, unroll=False)

    pltpu.make_async_copy(h_hbm.at[cid].at[idxb0.at[0]], hrows0, sem_h0).wait()
    pltpu.make_async_copy(emb_hbm.at[cid, pl.ds(0, _C)], erows0, sem_e0).wait()
    pltpu.make_async_copy(h_hbm.at[cid].at[idxb1.at[0]], hrows1, sem_h1).wait()
    pltpu.make_async_copy(emb_hbm.at[cid, pl.ds(0, _C)], erows1, sem_e1).wait()
    plsc.subcore_barrier()

    nb = sid * _NPT
    pltpu.sync_copy(SW.at[pl.ds(nb, _NPT)], out_hbm.at[cid, pl.ds(nb, _NPT)])


def _conv_sc(h_split, emb_split, idx_packed):
    kern = pl.kernel(
        _conv_body,
        out_type=jax.ShapeDtypeStruct((2, N, H), jnp.float32),
        mesh=_mesh,
        scratch_types=[
            pltpu.VMEM((2, _C), jnp.int32),
            pltpu.VMEM((2, _C), jnp.int32),
            pltpu.VMEM((_C, H2), jnp.float32),
            pltpu.VMEM((_C, H2), jnp.float32),
            pltpu.VMEM((_C, H2), jnp.float32),
            pltpu.VMEM((_C, H2), jnp.float32),
            pltpu.VMEM((_C, H), jnp.float32),
            pltpu.VMEM((_C, H), jnp.float32),
            pltpu.VMEM_SHARED((N, H), jnp.float32),
            pltpu.SemaphoreType.DMA,
            pltpu.SemaphoreType.DMA,
            pltpu.SemaphoreType.DMA,
            pltpu.SemaphoreType.DMA,
        ],
        compiler_params=pltpu.CompilerParams(use_tc_tiling_on_sc=False),
    )
    return kern(h_split, emb_split, idx_packed)


def kernel(x, edge_index, edge_attr, W_node, b_node, W_edge, b_edge,
           Wc0, bc0, Wc1, bc1, g0, be0, g1, be1):
    idx_packed = edge_index.reshape(2, _NCH, _C).transpose(1, 0, 2)
    b_node = b_node.reshape(1, H)
    b_edge = b_edge.reshape(1, H)
    bc0 = bc0.reshape(1, H)
    bc1 = bc1.reshape(1, H)
    g0 = g0.reshape(1, H)
    g1 = g1.reshape(1, H)
    be0 = be0.reshape(1, H)
    be1 = be1.reshape(1, H)

    h0 = _enc_node(x, W_node, b_node)
    emb = _enc_edge(edge_attr, W_edge, b_edge)
    a1 = _conv_sc(h0, emb, idx_packed)
    h2 = _mlp(h0, a1, Wc0, bc0, g0, be0, relu_out=True)
    a2 = _conv_sc(h2, emb, idx_packed)
    return _mlp(h2, a2, Wc1, bc1, g1, be1, relu_out=False)


# speedup vs baseline: 1.9685x; 1.9685x over previous
import functools

import jax
import jax.numpy as jnp
from jax import lax
from jax.experimental import pallas as pl
from jax.experimental.pallas import tpu as pltpu
from jax.experimental.pallas import tpu_sc as plsc

N = 10000
E = 320000
H = 128
H2 = 64
EPS = 1e-07


_BN = 2000
_BE = 4000


def _enc_node_body(x_ref, w_ref, b_ref, out_ref):
    h = jnp.dot(x_ref[...], w_ref[...], preferred_element_type=jnp.float32)
    h = h + b_ref[...]
    out_ref[0] = h[:, :H2]
    out_ref[1] = h[:, H2:]


def _enc_node(x, W, b):
    return pl.pallas_call(
        _enc_node_body,
        grid=(N // _BN,),
        in_specs=[
            pl.BlockSpec((_BN, 3), lambda i: (i, 0)),
            pl.BlockSpec((3, H), lambda i: (0, 0)),
            pl.BlockSpec((1, H), lambda i: (0, 0)),
        ],
        out_specs=pl.BlockSpec((2, _BN, H2), lambda i: (0, i, 0)),
        out_shape=jax.ShapeDtypeStruct((2, N, H2), jnp.float32),
    )(x, W, b)


def _enc_edge_body(a_ref, w_ref, b_ref, out_ref):
    h = jnp.dot(a_ref[...], w_ref[...], preferred_element_type=jnp.float32)
    h = h + b_ref[...]
    out_ref[0] = h[:, :H2]
    out_ref[1] = h[:, H2:]


def _enc_edge(attr, W, b):
    return pl.pallas_call(
        _enc_edge_body,
        grid=(E // _BE,),
        in_specs=[
            pl.BlockSpec((_BE, 7), lambda i: (i, 0)),
            pl.BlockSpec((7, H), lambda i: (0, 0)),
            pl.BlockSpec((1, H), lambda i: (0, 0)),
        ],
        out_specs=pl.BlockSpec((2, _BE, H2), lambda i: (0, i, 0)),
        out_shape=jax.ShapeDtypeStruct((2, E, H2), jnp.float32),
    )(attr, W, b)


def _mlp_body(relu_out, h_ref, s_ref, w_ref2, w_ref, b_ref, g_ref, be_ref,
              out_ref):
    a0 = w_ref2[0] / (s_ref[0] + 1e-16)
    a1 = w_ref2[1] / (s_ref[1] + 1e-16)
    hp = jnp.concatenate([h_ref[0] + a0, h_ref[1] + a1], axis=-1)
    t = jnp.dot(hp, w_ref[...], preferred_element_type=jnp.float32)
    t = t + b_ref[...]
    mu = jnp.mean(t, axis=-1, keepdims=True)
    var = jnp.mean((t - mu) * (t - mu), axis=-1, keepdims=True)
    y = (t - mu) / jnp.sqrt(var + 1e-5) * g_ref[...] + be_ref[...]
    if relu_out:
        y = jnp.maximum(y, 0.0)
        out_ref[0] = y[:, :H2]
        out_ref[1] = y[:, H2:]
    else:
        out_ref[...] = y


def _mlp(hs, aggr_s, aggr_w, Wc, bc, g, be, relu_out):
    if relu_out:
        out_spec = pl.BlockSpec((2, _BN, H2), lambda i: (0, i, 0))
        out_shape = jax.ShapeDtypeStruct((2, N, H2), jnp.float32)
    else:
        out_spec = pl.BlockSpec((_BN, H), lambda i: (i, 0))
        out_shape = jax.ShapeDtypeStruct((N, H), jnp.float32)
    return pl.pallas_call(
        functools.partial(_mlp_body, relu_out),
        grid=(N // _BN,),
        in_specs=[
            pl.BlockSpec((2, _BN, H2), lambda i: (0, i, 0)),
            pl.BlockSpec((2, _BN, H2), lambda i: (0, i, 0)),
            pl.BlockSpec((2, _BN, H2), lambda i: (0, i, 0)),
            pl.BlockSpec((H, H), lambda i: (0, 0)),
            pl.BlockSpec((1, H), lambda i: (0, 0)),
            pl.BlockSpec((1, H), lambda i: (0, 0)),
            pl.BlockSpec((1, H), lambda i: (0, 0)),
        ],
        out_specs=out_spec,
        out_shape=out_shape,
    )(hs, aggr_s, aggr_w, Wc, bc, g, be)


_NSUB = 16
_C = 80
_NCH = E // _C
_CPT = _NCH // _NSUB
_NPT = N // _NSUB
_FC = 25

_mesh = plsc.VectorSubcoreMesh(core_axis_name="c", subcore_axis_name="s")


_EPT = E // _NSUB
_CF = 128
_NFULL = _EPT // _CF
_CTL = _EPT - _NFULL * _CF


def _conv_body(h_hbm, emb_hbm, src_hbm, dst_hbm, outS_hbm, outW_hbm,
               srcv, dstv, srcvt, dstvt, hrows, erows, ebuf, wbuf,
               S_sh, W_sh, sem, sem2):
    cid = lax.axis_index("c")
    sid = lax.axis_index("s")

    zero = jnp.zeros((16,), jnp.float32)

    def zbody(e, carry):
        for k in range(4):
            ebuf[e, pl.ds(k * 16, 16)] = zero
        return carry

    lax.fori_loop(0, _CF, zbody, 0, unroll=False)
    for j in range(5):
        nb = sid * _NPT + j * 125
        pltpu.sync_copy(ebuf.at[pl.ds(0, 125)], S_sh.at[pl.ds(nb, 125)])
        pltpu.sync_copy(ebuf.at[pl.ds(0, 125)], W_sh.at[pl.ds(nb, 125)])
    plsc.subcore_barrier()

    def process(off, csz, sv, dv):
        pltpu.sync_copy(src_hbm.at[pl.ds(off, csz)], sv)
        pltpu.sync_copy(dst_hbm.at[pl.ds(off, csz)], dv)
        pltpu.async_copy(h_hbm.at[cid].at[sv], hrows.at[pl.ds(0, csz)], sem)
        pltpu.async_copy(emb_hbm.at[cid, pl.ds(off, csz)],
                         erows.at[pl.ds(0, csz)], sem2)
        pltpu.make_async_copy(h_hbm.at[cid].at[sv], hrows.at[pl.ds(0, csz)],
                              sem).wait()
        pltpu.make_async_copy(emb_hbm.at[cid, pl.ds(off, csz)],
                              erows.at[pl.ds(0, csz)], sem2).wait()

        def cbody(e, carry):
            for k in range(4):
                sl = pl.ds(k * 16, 16)
                msg = jnp.maximum(hrows[e, sl] + erows[e, sl], 0.0) + EPS
                ex = jnp.exp(msg)
                ebuf[e, sl] = ex
                wbuf[e, sl] = ex * msg
            return carry

        lax.fori_loop(0, csz, cbody, 0, unroll=False)
        pltpu.sync_copy(ebuf.at[pl.ds(0, csz)], S_sh.at[dv], add=True)
        pltpu.sync_copy(wbuf.at[pl.ds(0, csz)], W_sh.at[dv], add=True)

    base = sid * _EPT

    def chunk(i, carry):
        process(base + i * _CF, _CF, srcv, dstv)
        return carry

    lax.fori_loop(0, _NFULL, chunk, 0, unroll=False)
    if _CTL:
        process(base + _NFULL * _CF, _CTL, srcvt, dstvt)

    plsc.subcore_barrier()

    nb = sid * _NPT
    pltpu.sync_copy(S_sh.at[pl.ds(nb, _NPT)], outS_hbm.at[cid, pl.ds(nb, _NPT)])
    pltpu.sync_copy(W_sh.at[pl.ds(nb, _NPT)], outW_hbm.at[cid, pl.ds(nb, _NPT)])


def _conv_sc(h_split, emb_split, src, dst):
    kern = pl.kernel(
        _conv_body,
        out_type=[jax.ShapeDtypeStruct((2, N, H2), jnp.float32),
                  jax.ShapeDtypeStruct((2, N, H2), jnp.float32)],
        mesh=_mesh,
        scratch_types=[
            pltpu.VMEM((_CF,), jnp.int32),
            pltpu.VMEM((_CF,), jnp.int32),
            pltpu.VMEM((_CTL,), jnp.int32),
            pltpu.VMEM((_CTL,), jnp.int32),
            pltpu.VMEM((_CF, H2), jnp.float32),
            pltpu.VMEM((_CF, H2), jnp.float32),
            pltpu.VMEM((_CF, H2), jnp.float32),
            pltpu.VMEM((_CF, H2), jnp.float32),
            pltpu.VMEM_SHARED((N, H2), jnp.float32),
            pltpu.VMEM_SHARED((N, H2), jnp.float32),
            pltpu.SemaphoreType.DMA,
            pltpu.SemaphoreType.DMA,
        ],
        compiler_params=pltpu.CompilerParams(use_tc_tiling_on_sc=False),
    )
    return kern(h_split, emb_split, src, dst)


def kernel(x, edge_index, edge_attr, W_node, b_node, W_edge, b_edge,
           Wc0, bc0, Wc1, bc1, g0, be0, g1, be1):
    src = edge_index[0]
    dst = edge_index[1]
    b_node = b_node.reshape(1, H)
    b_edge = b_edge.reshape(1, H)
    bc0 = bc0.reshape(1, H)
    bc1 = bc1.reshape(1, H)
    g0 = g0.reshape(1, H)
    g1 = g1.reshape(1, H)
    be0 = be0.reshape(1, H)
    be1 = be1.reshape(1, H)

    h0 = _enc_node(x, W_node, b_node)
    emb = _enc_edge(edge_attr, W_edge, b_edge)
    s1, w1 = _conv_sc(h0, emb, src, dst)
    h2 = _mlp(h0, s1, w1, Wc0, bc0, g0, be0, relu_out=True)
    s2, w2 = _conv_sc(h2, emb, src, dst)
    return _mlp(h2, s2, w2, Wc1, bc1, g1, be1, relu_out=False)


# speedup vs baseline: 2.9098x; 1.4782x over previous
import functools

import jax
import jax.numpy as jnp
from jax import lax
from jax.experimental import pallas as pl
from jax.experimental.pallas import tpu as pltpu
from jax.experimental.pallas import tpu_sc as plsc

N = 10000
E = 320000
H = 128
H2 = 64
EPS = 1e-07


_BN = 2000
_BE = 4000


def _enc_node_body(x_ref, w_ref, b_ref, out_ref):
    h = jnp.dot(x_ref[...], w_ref[...], preferred_element_type=jnp.float32)
    h = h + b_ref[...]
    out_ref[0] = h[:, :H2]
    out_ref[1] = h[:, H2:]


def _enc_node(x, W, b):
    return pl.pallas_call(
        _enc_node_body,
        grid=(N // _BN,),
        in_specs=[
            pl.BlockSpec((_BN, 3), lambda i: (i, 0)),
            pl.BlockSpec((3, H), lambda i: (0, 0)),
            pl.BlockSpec((1, H), lambda i: (0, 0)),
        ],
        out_specs=pl.BlockSpec((2, _BN, H2), lambda i: (0, i, 0)),
        out_shape=jax.ShapeDtypeStruct((2, N, H2), jnp.float32),
    )(x, W, b)


def _enc_edge_body(a_ref, w_ref, b_ref, out_ref):
    h = jnp.dot(a_ref[...], w_ref[...], preferred_element_type=jnp.float32)
    h = h + b_ref[...]
    out_ref[0] = h[:, :H2]
    out_ref[1] = h[:, H2:]


def _enc_edge(attr, W, b):
    return pl.pallas_call(
        _enc_edge_body,
        grid=(E // _BE,),
        in_specs=[
            pl.BlockSpec((_BE, 7), lambda i: (i, 0)),
            pl.BlockSpec((7, H), lambda i: (0, 0)),
            pl.BlockSpec((1, H), lambda i: (0, 0)),
        ],
        out_specs=pl.BlockSpec((2, _BE, H2), lambda i: (0, i, 0)),
        out_shape=jax.ShapeDtypeStruct((2, E, H2), jnp.float32),
    )(attr, W, b)


def _mlp_body(relu_out, h_ref, s_ref, w_ref2, w_ref, b_ref, g_ref, be_ref,
              out_ref):
    a0 = w_ref2[0] / (s_ref[0] + 1e-16)
    a1 = w_ref2[1] / (s_ref[1] + 1e-16)
    hp = jnp.concatenate([h_ref[0] + a0, h_ref[1] + a1], axis=-1)
    t = jnp.dot(hp, w_ref[...], preferred_element_type=jnp.float32)
    t = t + b_ref[...]
    mu = jnp.mean(t, axis=-1, keepdims=True)
    var = jnp.mean((t - mu) * (t - mu), axis=-1, keepdims=True)
    y = (t - mu) / jnp.sqrt(var + 1e-5) * g_ref[...] + be_ref[...]
    if relu_out:
        y = jnp.maximum(y, 0.0)
        out_ref[0] = y[:, :H2]
        out_ref[1] = y[:, H2:]
    else:
        out_ref[...] = y


def _mlp(hs, aggr_s, aggr_w, Wc, bc, g, be, relu_out):
    if relu_out:
        out_spec = pl.BlockSpec((2, _BN, H2), lambda i: (0, i, 0))
        out_shape = jax.ShapeDtypeStruct((2, N, H2), jnp.float32)
    else:
        out_spec = pl.BlockSpec((_BN, H), lambda i: (i, 0))
        out_shape = jax.ShapeDtypeStruct((N, H), jnp.float32)
    return pl.pallas_call(
        functools.partial(_mlp_body, relu_out),
        grid=(N // _BN,),
        in_specs=[
            pl.BlockSpec((2, _BN, H2), lambda i: (0, i, 0)),
            pl.BlockSpec((2, _BN, H2), lambda i: (0, i, 0)),
            pl.BlockSpec((2, _BN, H2), lambda i: (0, i, 0)),
            pl.BlockSpec((H, H), lambda i: (0, 0)),
            pl.BlockSpec((1, H), lambda i: (0, 0)),
            pl.BlockSpec((1, H), lambda i: (0, 0)),
            pl.BlockSpec((1, H), lambda i: (0, 0)),
        ],
        out_specs=out_spec,
        out_shape=out_shape,
    )(hs, aggr_s, aggr_w, Wc, bc, g, be)


_NSUB = 16
_C = 80
_NCH = E // _C
_CPT = _NCH // _NSUB
_NPT = N // _NSUB
_FC = 25

_mesh = plsc.VectorSubcoreMesh(core_axis_name="c", subcore_axis_name="s")


_CG = 100
_NCHG = E // _CG
_CPTG = _NCHG // _NSUB
_G = 20
_NG = _CPTG // _G


def _conv_body(h_hbm, emb_hbm, idx_hbm, outS_hbm, outW_hbm,
               idxg, hrows0, hrows1, erows0, erows1, ebuf, wbuf,
               S_sh, W_sh, sem_h0, sem_h1, sem_e0, sem_e1):
    cid = lax.axis_index("c")
    sid = lax.axis_index("s")

    zero = jnp.zeros((16,), jnp.float32)

    def zbody(e, carry):
        for k in range(4):
            ebuf[e, pl.ds(k * 16, 16)] = zero
        return carry

    lax.fori_loop(0, _CG, zbody, 0, unroll=False)
    for j in range(7):
        nb = sid * _NPT + j * 90
        sz = 90 if j < 6 else 85
        pltpu.sync_copy(ebuf.at[pl.ds(0, sz)], S_sh.at[pl.ds(nb, sz)])
        pltpu.sync_copy(ebuf.at[pl.ds(0, sz)], W_sh.at[pl.ds(nb, sz)])
    plsc.subcore_barrier()

    hr = (hrows0, hrows1)
    er = (erows0, erows1)
    sh = (sem_h0, sem_h1)
    se = (sem_e0, sem_e1)

    def group(g, carry):
        cb = sid * _CPTG + g * _G
        pltpu.sync_copy(idx_hbm.at[pl.ds(cb, _G)], idxg)

        def issue(j):
            s = j % 2
            pltpu.async_copy(h_hbm.at[cid].at[idxg.at[j, 0]], hr[s], sh[s])
            pltpu.async_copy(emb_hbm.at[cid, pl.ds((cb + j) * _CG, _CG)],
                             er[s], se[s])

        issue(0)
        for j in range(_G):
            s = j % 2
            if j + 1 < _G:
                issue(j + 1)
            pltpu.make_async_copy(h_hbm.at[cid].at[idxg.at[j, 0]],
                                  hr[s], sh[s]).wait()
            pltpu.make_async_copy(emb_hbm.at[cid, pl.ds(0, _CG)],
                                  er[s], se[s]).wait()

            def cbody(e, carry2):
                for k in range(4):
                    sl = pl.ds(k * 16, 16)
                    msg = jnp.maximum(hr[s][e, sl] + er[s][e, sl], 0.0) + EPS
                    ex = jnp.exp(msg)
                    ebuf[e, sl] = ex
                    wbuf[e, sl] = ex * msg
                return carry2

            lax.fori_loop(0, _CG, cbody, 0, unroll=False)
            pltpu.sync_copy(ebuf, S_sh.at[idxg.at[j, 1]], add=True)
            pltpu.sync_copy(wbuf, W_sh.at[idxg.at[j, 1]], add=True)
        return carry

    lax.fori_loop(0, _NG, group, 0, unroll=False)
    plsc.subcore_barrier()

    nb = sid * _NPT
    pltpu.sync_copy(S_sh.at[pl.ds(nb, _NPT)], outS_hbm.at[cid, pl.ds(nb, _NPT)])
    pltpu.sync_copy(W_sh.at[pl.ds(nb, _NPT)], outW_hbm.at[cid, pl.ds(nb, _NPT)])


def _conv_sc(h_split, emb_split, idx_packed):
    kern = pl.kernel(
        _conv_body,
        out_type=[jax.ShapeDtypeStruct((2, N, H2), jnp.float32),
                  jax.ShapeDtypeStruct((2, N, H2), jnp.float32)],
        mesh=_mesh,
        scratch_types=[
            pltpu.VMEM((_G, 2, _CG), jnp.int32),
            pltpu.VMEM((_CG, H2), jnp.float32),
            pltpu.VMEM((_CG, H2), jnp.float32),
            pltpu.VMEM((_CG, H2), jnp.float32),
            pltpu.VMEM((_CG, H2), jnp.float32),
            pltpu.VMEM((_CG, H2), jnp.float32),
            pltpu.VMEM((_CG, H2), jnp.float32),
            pltpu.VMEM_SHARED((N, H2), jnp.float32),
            pltpu.VMEM_SHARED((N, H2), jnp.float32),
            pltpu.SemaphoreType.DMA,
            pltpu.SemaphoreType.DMA,
            pltpu.SemaphoreType.DMA,
            pltpu.SemaphoreType.DMA,
        ],
        compiler_params=pltpu.CompilerParams(use_tc_tiling_on_sc=False),
    )
    return kern(h_split, emb_split, idx_packed)


def kernel(x, edge_index, edge_attr, W_node, b_node, W_edge, b_edge,
           Wc0, bc0, Wc1, bc1, g0, be0, g1, be1):
    idx_packed = edge_index.reshape(2, _NCHG, _CG).transpose(1, 0, 2)
    b_node = b_node.reshape(1, H)
    b_edge = b_edge.reshape(1, H)
    bc0 = bc0.reshape(1, H)
    bc1 = bc1.reshape(1, H)
    g0 = g0.reshape(1, H)
    g1 = g1.reshape(1, H)
    be0 = be0.reshape(1, H)
    be1 = be1.reshape(1, H)

    h0 = _enc_node(x, W_node, b_node)
    emb = _enc_edge(edge_attr, W_edge, b_edge)
    s1, w1 = _conv_sc(h0, emb, idx_packed)
    h2 = _mlp(h0, s1, w1, Wc0, bc0, g0, be0, relu_out=True)
    s2, w2 = _conv_sc(h2, emb, idx_packed)
    return _mlp(h2, s2, w2, Wc1, bc1, g1, be1, relu_out=False)


# speedup vs baseline: 3.0979x; 1.0646x over previous
import functools

import jax
import jax.numpy as jnp
from jax import lax
from jax.experimental import pallas as pl
from jax.experimental.pallas import tpu as pltpu
from jax.experimental.pallas import tpu_sc as plsc

N = 10000
E = 320000
H = 128
H2 = 64
EPS = 1e-07


_BN = 2000
_BE = 4000


def _enc_node_body(x_ref, w_ref, b_ref, out_ref):
    h = jnp.dot(x_ref[...], w_ref[...], preferred_element_type=jnp.float32)
    h = h + b_ref[...]
    out_ref[0] = h[:, :H2]
    out_ref[1] = h[:, H2:]


def _enc_node(x, W, b):
    return pl.pallas_call(
        _enc_node_body,
        grid=(N // _BN,),
        in_specs=[
            pl.BlockSpec((_BN, 3), lambda i: (i, 0)),
            pl.BlockSpec((3, H), lambda i: (0, 0)),
            pl.BlockSpec((1, H), lambda i: (0, 0)),
        ],
        out_specs=pl.BlockSpec((2, _BN, H2), lambda i: (0, i, 0)),
        out_shape=jax.ShapeDtypeStruct((2, N, H2), jnp.float32),
    )(x, W, b)


def _enc_edge_body(a_ref, w_ref, b_ref, out_ref):
    h = jnp.dot(a_ref[...], w_ref[...], preferred_element_type=jnp.float32)
    h = h + b_ref[...]
    out_ref[0] = h[:, :H2]
    out_ref[1] = h[:, H2:]


def _enc_edge(attr, W, b):
    return pl.pallas_call(
        _enc_edge_body,
        grid=(E // _BE,),
        in_specs=[
            pl.BlockSpec((_BE, 7), lambda i: (i, 0)),
            pl.BlockSpec((7, H), lambda i: (0, 0)),
            pl.BlockSpec((1, H), lambda i: (0, 0)),
        ],
        out_specs=pl.BlockSpec((2, _BE, H2), lambda i: (0, i, 0)),
        out_shape=jax.ShapeDtypeStruct((2, E, H2), jnp.float32),
    )(attr, W, b)


def _mlp_body(relu_out, h_ref, s_ref, w_ref2, w_ref, b_ref, g_ref, be_ref,
              out_ref):
    a0 = w_ref2[0] / (s_ref[0] + 1e-16)
    a1 = w_ref2[1] / (s_ref[1] + 1e-16)
    hp = jnp.concatenate([h_ref[0] + a0, h_ref[1] + a1], axis=-1)
    t = jnp.dot(hp, w_ref[...], preferred_element_type=jnp.float32)
    t = t + b_ref[...]
    mu = jnp.mean(t, axis=-1, keepdims=True)
    var = jnp.mean((t - mu) * (t - mu), axis=-1, keepdims=True)
    y = (t - mu) / jnp.sqrt(var + 1e-5) * g_ref[...] + be_ref[...]
    if relu_out:
        y = jnp.maximum(y, 0.0)
        out_ref[0] = y[:, :H2]
        out_ref[1] = y[:, H2:]
    else:
        out_ref[...] = y


def _mlp(hs, aggr_s, aggr_w, Wc, bc, g, be, relu_out):
    if relu_out:
        out_spec = pl.BlockSpec((2, _BN, H2), lambda i: (0, i, 0))
        out_shape = jax.ShapeDtypeStruct((2, N, H2), jnp.float32)
    else:
        out_spec = pl.BlockSpec((_BN, H), lambda i: (i, 0))
        out_shape = jax.ShapeDtypeStruct((N, H), jnp.float32)
    return pl.pallas_call(
        functools.partial(_mlp_body, relu_out),
        grid=(N // _BN,),
        in_specs=[
            pl.BlockSpec((2, _BN, H2), lambda i: (0, i, 0)),
            pl.BlockSpec((2, _BN, H2), lambda i: (0, i, 0)),
            pl.BlockSpec((2, _BN, H2), lambda i: (0, i, 0)),
            pl.BlockSpec((H, H), lambda i: (0, 0)),
            pl.BlockSpec((1, H), lambda i: (0, 0)),
            pl.BlockSpec((1, H), lambda i: (0, 0)),
            pl.BlockSpec((1, H), lambda i: (0, 0)),
        ],
        out_specs=out_spec,
        out_shape=out_shape,
    )(hs, aggr_s, aggr_w, Wc, bc, g, be)


_NSUB = 16
_C = 80
_NCH = E // _C
_CPT = _NCH // _NSUB
_NPT = N // _NSUB
_FC = 25

_mesh = plsc.VectorSubcoreMesh(core_axis_name="c", subcore_axis_name="s")


_CG = 80
_NCHG = E // _CG
_CPTG = _NCHG // _NSUB
_G = 25
_NG = _CPTG // _G


def _conv_body(h_hbm, emb_hbm, idx_hbm, outS_hbm, outW_hbm,
               idxg, hrows0, hrows1, erows0, erows1, ebuf0, ebuf1,
               wbuf0, wbuf1, S_sh, W_sh,
               sem_h0, sem_h1, sem_e0, sem_e1, sem_s0, sem_s1):
    cid = lax.axis_index("c")
    sid = lax.axis_index("s")

    zero = jnp.zeros((16,), jnp.float32)

    def zbody(e, carry):
        for k in range(4):
            ebuf0[e, pl.ds(k * 16, 16)] = zero
        return carry

    lax.fori_loop(0, _CG, zbody, 0, unroll=False)
    for j in range(8):
        nb = sid * _NPT + j * 80
        sz = 80 if j < 7 else 65
        pltpu.sync_copy(ebuf0.at[pl.ds(0, sz)], S_sh.at[pl.ds(nb, sz)])
        pltpu.sync_copy(ebuf0.at[pl.ds(0, sz)], W_sh.at[pl.ds(nb, sz)])
    plsc.subcore_barrier()

    hr = (hrows0, hrows1)
    er = (erows0, erows1)
    eb = (ebuf0, ebuf1)
    wb = (wbuf0, wbuf1)
    sh = (sem_h0, sem_h1)
    se = (sem_e0, sem_e1)
    ss = (sem_s0, sem_s1)

    def group(g, carry):
        cb = sid * _CPTG + g * _G
        pltpu.sync_copy(idx_hbm.at[pl.ds(cb, _G)], idxg)

        def issue(j):
            s = j % 2
            pltpu.async_copy(h_hbm.at[cid].at[idxg.at[j, 0]], hr[s], sh[s])
            pltpu.async_copy(emb_hbm.at[cid, pl.ds((cb + j) * _CG, _CG)],
                             er[s], se[s])

        def drain_scatter(s):
            pltpu.make_async_copy(eb[s], S_sh.at[idxg.at[0, 1]], ss[s]).wait()
            pltpu.make_async_copy(wb[s], W_sh.at[idxg.at[0, 1]], ss[s]).wait()

        issue(0)
        for j in range(_G):
            s = j % 2
            if j + 1 < _G:
                issue(j + 1)
            pltpu.make_async_copy(h_hbm.at[cid].at[idxg.at[j, 0]],
                                  hr[s], sh[s]).wait()
            pltpu.make_async_copy(emb_hbm.at[cid, pl.ds(0, _CG)],
                                  er[s], se[s]).wait()
            if j >= 2:
                drain_scatter(s)

            def cbody(e, carry2):
                for k in range(4):
                    sl = pl.ds(k * 16, 16)
                    msg = jnp.maximum(hr[s][e, sl] + er[s][e, sl], 0.0) + EPS
                    ex = jnp.exp(msg)
                    eb[s][e, sl] = ex
                    wb[s][e, sl] = ex * msg
                return carry2

            lax.fori_loop(0, _CG, cbody, 0, unroll=False)
            pltpu.async_copy(eb[s], S_sh.at[idxg.at[j, 1]], ss[s], add=True)
            pltpu.async_copy(wb[s], W_sh.at[idxg.at[j, 1]], ss[s], add=True)
        drain_scatter((_G - 2) % 2)
        drain_scatter((_G - 1) % 2)
        return carry

    lax.fori_loop(0, _NG, group, 0, unroll=False)
    plsc.subcore_barrier()

    nb = sid * _NPT
    pltpu.sync_copy(S_sh.at[pl.ds(nb, _NPT)], outS_hbm.at[cid, pl.ds(nb, _NPT)])
    pltpu.sync_copy(W_sh.at[pl.ds(nb, _NPT)], outW_hbm.at[cid, pl.ds(nb, _NPT)])


def _conv_sc(h_split, emb_split, idx_packed):
    kern = pl.kernel(
        _conv_body,
        out_type=[jax.ShapeDtypeStruct((2, N, H2), jnp.float32),
                  jax.ShapeDtypeStruct((2, N, H2), jnp.float32)],
        mesh=_mesh,
        scratch_types=[
            pltpu.VMEM((_G, 2, _CG), jnp.int32),
            pltpu.VMEM((_CG, H2), jnp.float32),
            pltpu.VMEM((_CG, H2), jnp.float32),
            pltpu.VMEM((_CG, H2), jnp.float32),
            pltpu.VMEM((_CG, H2), jnp.float32),
            pltpu.VMEM((_CG, H2), jnp.float32),
            pltpu.VMEM((_CG, H2), jnp.float32),
            pltpu.VMEM((_CG, H2), jnp.float32),
            pltpu.VMEM((_CG, H2), jnp.float32),
            pltpu.VMEM_SHARED((N, H2), jnp.float32),
            pltpu.VMEM_SHARED((N, H2), jnp.float32),
            pltpu.SemaphoreType.DMA,
            pltpu.SemaphoreType.DMA,
            pltpu.SemaphoreType.DMA,
            pltpu.SemaphoreType.DMA,
            pltpu.SemaphoreType.DMA,
            pltpu.SemaphoreType.DMA,
        ],
        compiler_params=pltpu.CompilerParams(use_tc_tiling_on_sc=False),
    )
    return kern(h_split, emb_split, idx_packed)


def kernel(x, edge_index, edge_attr, W_node, b_node, W_edge, b_edge,
           Wc0, bc0, Wc1, bc1, g0, be0, g1, be1):
    idx_packed = edge_index.reshape(2, _NCHG, _CG).transpose(1, 0, 2)
    b_node = b_node.reshape(1, H)
    b_edge = b_edge.reshape(1, H)
    bc0 = bc0.reshape(1, H)
    bc1 = bc1.reshape(1, H)
    g0 = g0.reshape(1, H)
    g1 = g1.reshape(1, H)
    be0 = be0.reshape(1, H)
    be1 = be1.reshape(1, H)

    h0 = _enc_node(x, W_node, b_node)
    emb = _enc_edge(edge_attr, W_edge, b_edge)
    s1, w1 = _conv_sc(h0, emb, idx_packed)
    h2 = _mlp(h0, s1, w1, Wc0, bc0, g0, be0, relu_out=True)
    s2, w2 = _conv_sc(h2, emb, idx_packed)
    return _mlp(h2, s2, w2, Wc1, bc1, g1, be1, relu_out=False)
